# baseline (device time: 3233507 ns/iter reference)
import jax
import jax.numpy as jnp
from jax import lax
from jax.experimental import pallas as pl
from jax.experimental.pallas import tpu as pltpu

N_EXPERTS = 8
E_LOCAL = 4
C = 640
FC = 512
NSEM = 16


def _row_gather(src, idx):
    n_out = idx.shape[0]
    d = src.shape[1]

    def body(idx_ref, src_ref, out_ref, sems):
        def mk(i):
            r = idx_ref[i]
            return pltpu.make_async_copy(
                src_ref.at[pl.ds(r, 1), :],
                out_ref.at[pl.ds(i, 1), :],
                sems.at[i % NSEM],
            )

        def loop(i, carry):
            @pl.when(i >= NSEM)
            def _():
                mk(i - NSEM).wait()

            mk(i).start()
            return carry

        lax.fori_loop(0, n_out, loop, 0)

        def drain(i, carry):
            mk(n_out - NSEM + i).wait()
            return carry

        lax.fori_loop(0, NSEM, drain, 0)

    return pl.pallas_call(
        body,
        in_specs=[
            pl.BlockSpec(memory_space=pltpu.SMEM),
            pl.BlockSpec(memory_space=pl.ANY),
        ],
        out_specs=pl.BlockSpec(memory_space=pl.ANY),
        out_shape=jax.ShapeDtypeStruct((n_out, d), src.dtype),
        scratch_shapes=[pltpu.SemaphoreType.DMA((NSEM,))],
    )(idx, src)


def _a2a_exchange(buf, collective_id):

    def body(b_ref, o_ref, send_sem, recv_sem):
        my_x = lax.axis_index("x")
        my_y = lax.axis_index("y")
        my_z = lax.axis_index("z")
        partner = (1 - my_x, my_y, my_z)

        barrier = pltpu.get_barrier_semaphore()
        pl.semaphore_signal(
            barrier, inc=1, device_id=partner,
            device_id_type=pl.DeviceIdType.MESH,
        )
        pl.semaphore_wait(barrier, 1)

        rdma = pltpu.make_async_remote_copy(
            src_ref=b_ref,
            dst_ref=o_ref,
            send_sem=send_sem,
            recv_sem=recv_sem,
            device_id=partner,
            device_id_type=pl.DeviceIdType.MESH,
        )
        rdma.start()
        rdma.wait()

    return pl.pallas_call(
        body,
        out_shape=jax.ShapeDtypeStruct(buf.shape, buf.dtype),
        in_specs=[pl.BlockSpec(memory_space=pltpu.VMEM)],
        out_specs=pl.BlockSpec(memory_space=pltpu.VMEM),
        scratch_shapes=[pltpu.SemaphoreType.DMA, pltpu.SemaphoreType.DMA],
        compiler_params=pltpu.CompilerParams(collective_id=collective_id),
    )(buf)


def _expert_ffn(xcat, W1, W2):
    n_e, rows, d = xcat.shape
    f = W1.shape[2]
    nk = f // FC
    R = 640
    nr = rows // R

    def body(x_ref, w1_ref, w2_ref, o_ref):
        k = pl.program_id(2)
        h = jnp.maximum(
            jnp.dot(x_ref[0], w1_ref[0], preferred_element_type=jnp.float32),
            0.0,
        )
        contrib = jnp.dot(h, w2_ref[0], preferred_element_type=jnp.float32)

        @pl.when(k == 0)
        def _():
            o_ref[0] = contrib

        @pl.when(k != 0)
        def _():
            o_ref[0] += contrib

    return pl.pallas_call(
        body,
        grid=(n_e, nr, nk),
        in_specs=[
            pl.BlockSpec((1, R, d), lambda e, r, k: (e, r, 0)),
            pl.BlockSpec((1, d, FC), lambda e, r, k: (e, 0, k)),
            pl.BlockSpec((1, FC, d), lambda e, r, k: (e, k, 0)),
        ],
        out_specs=pl.BlockSpec((1, R, d), lambda e, r, k: (e, r, 0)),
        out_shape=jax.ShapeDtypeStruct((n_e, rows, d), jnp.float32),
        compiler_params=pltpu.CompilerParams(
            vmem_limit_bytes=56 * 1024 * 1024
        ),
    )(xcat, W1, W2)


def kernel(x, assign, W1, W2):
    T, D = x.shape
    my_x = lax.axis_index("x")
    my_base = my_x * E_LOCAL
    partner_base = (1 - my_x) * E_LOCAL

    idx_all = jnp.stack(
        [jnp.nonzero(assign == e, size=C, fill_value=T)[0]
         for e in range(N_EXPERTS)]
    )
    idx_mine = lax.dynamic_slice_in_dim(idx_all, my_base, E_LOCAL, axis=0)
    idx_sent = lax.dynamic_slice_in_dim(idx_all, partner_base, E_LOCAL, axis=0)
    idx_ord = jnp.concatenate([idx_mine, idx_sent]).reshape(-1)

    xp = jnp.concatenate([x, jnp.zeros((1, D), x.dtype)], axis=0)
    gathered = _row_gather(xp, idx_ord)
    local_buf = gathered[: E_LOCAL * C].reshape(E_LOCAL, C, D)
    send_buf = gathered[E_LOCAL * C:].reshape(E_LOCAL, C, D)

    recv_buf = _a2a_exchange(send_buf, collective_id=0)

    xcat = jnp.concatenate([local_buf, recv_buf], axis=1)
    y = _expert_ffn(xcat, W1, W2)
    y_local = y[:, :C]
    y_remote = y[:, C:]

    y_back = _a2a_exchange(y_remote, collective_id=1)

    y_all = jnp.concatenate([y_local, y_back]).reshape(-1, D)
    inv = jnp.zeros((T + 1,), jnp.int32).at[idx_ord].set(
        jnp.arange(idx_ord.shape[0], dtype=jnp.int32)
    )
    return _row_gather(y_all, inv[:T])


# device time: 1100943 ns/iter; 2.9370x vs baseline; 2.9370x over previous
import jax
import jax.numpy as jnp
from jax import lax
from jax.experimental import pallas as pl
from jax.experimental.pallas import tpu as pltpu

N_EXPERTS = 8
E_LOCAL = 4
C = 640
FC = 512
_VMEM = pltpu.CompilerParams(vmem_limit_bytes=56 * 1024 * 1024)


def _permute_to_slots(x, inv_row, n_slots):
    T, D = x.shape
    R, KC = 640, 1024

    def body(inv_ref, x_ref, o_ref):
        r, k = pl.program_id(0), pl.program_id(1)
        rows = r * R + lax.broadcasted_iota(jnp.int32, (R, KC), 0)
        P = (inv_ref[...] == rows).astype(jnp.float32)
        contrib = jnp.dot(P, x_ref[...], preferred_element_type=jnp.float32)

        @pl.when(k == 0)
        def _():
            o_ref[...] = contrib

        @pl.when(k != 0)
        def _():
            o_ref[...] += contrib

    return pl.pallas_call(
        body,
        grid=(n_slots // R, T // KC),
        in_specs=[
            pl.BlockSpec((1, KC), lambda r, k: (0, k)),
            pl.BlockSpec((KC, D), lambda r, k: (k, 0)),
        ],
        out_specs=pl.BlockSpec((R, D), lambda r, k: (r, 0)),
        out_shape=jax.ShapeDtypeStruct((n_slots, D), x.dtype),
        compiler_params=_VMEM,
    )(inv_row, x)


def _gather_from_slots(y, inv_col, T):
    S, D = y.shape
    R, KC = 512, 1280

    def body(inv_ref, y_ref, o_ref):
        k = pl.program_id(1)
        cols = k * KC + lax.broadcasted_iota(jnp.int32, (R, KC), 1)
        P = (inv_ref[...] == cols).astype(jnp.float32)
        contrib = jnp.dot(P, y_ref[...], preferred_element_type=jnp.float32)

        @pl.when(k == 0)
        def _():
            o_ref[...] = contrib

        @pl.when(k != 0)
        def _():
            o_ref[...] += contrib

    return pl.pallas_call(
        body,
        grid=(T // R, S // KC),
        in_specs=[
            pl.BlockSpec((R, 1), lambda r, k: (r, 0)),
            pl.BlockSpec((KC, D), lambda r, k: (k, 0)),
        ],
        out_specs=pl.BlockSpec((R, D), lambda r, k: (r, 0)),
        out_shape=jax.ShapeDtypeStruct((T, D), y.dtype),
        compiler_params=_VMEM,
    )(inv_col, y)


def _a2a_exchange(buf, collective_id):

    def body(b_ref, o_ref, send_sem, recv_sem):
        my_x = lax.axis_index("x")
        my_y = lax.axis_index("y")
        my_z = lax.axis_index("z")
        partner = (1 - my_x, my_y, my_z)

        barrier = pltpu.get_barrier_semaphore()
        pl.semaphore_signal(
            barrier, inc=1, device_id=partner,
            device_id_type=pl.DeviceIdType.MESH,
        )
        pl.semaphore_wait(barrier, 1)

        rdma = pltpu.make_async_remote_copy(
            src_ref=b_ref,
            dst_ref=o_ref,
            send_sem=send_sem,
            recv_sem=recv_sem,
            device_id=partner,
            device_id_type=pl.DeviceIdType.MESH,
        )
        rdma.start()
        rdma.wait()

    return pl.pallas_call(
        body,
        out_shape=jax.ShapeDtypeStruct(buf.shape, buf.dtype),
        in_specs=[pl.BlockSpec(memory_space=pltpu.VMEM)],
        out_specs=pl.BlockSpec(memory_space=pltpu.VMEM),
        scratch_shapes=[pltpu.SemaphoreType.DMA, pltpu.SemaphoreType.DMA],
        compiler_params=pltpu.CompilerParams(collective_id=collective_id),
    )(buf)


def _expert_ffn(xcat, W1, W2):
    n_e, rows, d = xcat.shape
    f = W1.shape[2]
    nk = f // FC
    R = 640
    nr = rows // R

    def body(x_ref, w1_ref, w2_ref, o_ref):
        k = pl.program_id(2)
        h = jnp.maximum(
            jnp.dot(x_ref[0], w1_ref[0], preferred_element_type=jnp.float32),
            0.0,
        )
        contrib = jnp.dot(h, w2_ref[0], preferred_element_type=jnp.float32)

        @pl.when(k == 0)
        def _():
            o_ref[0] = contrib

        @pl.when(k != 0)
        def _():
            o_ref[0] += contrib

    return pl.pallas_call(
        body,
        grid=(n_e, nr, nk),
        in_specs=[
            pl.BlockSpec((1, R, d), lambda e, r, k: (e, r, 0)),
            pl.BlockSpec((1, d, FC), lambda e, r, k: (e, 0, k)),
            pl.BlockSpec((1, FC, d), lambda e, r, k: (e, k, 0)),
        ],
        out_specs=pl.BlockSpec((1, R, d), lambda e, r, k: (e, r, 0)),
        out_shape=jax.ShapeDtypeStruct((n_e, rows, d), jnp.float32),
        compiler_params=_VMEM,
    )(xcat, W1, W2)


def kernel(x, assign, W1, W2):
    T, D = x.shape
    my_x = lax.axis_index("x")
    my_base = my_x * E_LOCAL

    oh = (assign[:, None] == jnp.arange(N_EXPERTS)[None, :]).astype(jnp.int32)
    rank = (oh * (jnp.cumsum(oh, axis=0) - oh)).sum(axis=1)
    slot_e = jnp.remainder(assign - my_base, N_EXPERTS)
    inv = (slot_e * C + rank).astype(jnp.int32)

    nloc = E_LOCAL * C
    gathered = _permute_to_slots(x, inv.reshape(1, T), 2 * nloc)
    local_buf = gathered[:nloc].reshape(E_LOCAL, C, D)
    send_buf = gathered[nloc:].reshape(E_LOCAL, C, D)

    recv_buf = _a2a_exchange(send_buf, collective_id=0)

    xcat = jnp.concatenate([local_buf, recv_buf], axis=1)
    y = _expert_ffn(xcat, W1, W2)
    y_local = y[:, :C]
    y_remote = y[:, C:]

    y_back = _a2a_exchange(y_remote, collective_id=1)

    y_all = jnp.concatenate([y_local, y_back]).reshape(-1, D)
    return _gather_from_slots(y_all, inv.reshape(T, 1), T)


# device time: 785282 ns/iter; 4.1176x vs baseline; 1.4020x over previous
import jax
import jax.numpy as jnp
from jax import lax
from jax.experimental import pallas as pl
from jax.experimental.pallas import tpu as pltpu

N_EXPERTS = 8
E_LOCAL = 4
C = 640
FC = 512
_VMEM = pltpu.CompilerParams(vmem_limit_bytes=56 * 1024 * 1024)


def _permute_to_slots(x, inv_row, n_slots):
    T, D = x.shape
    R, KC = 640, 1024

    def body(inv_ref, x_ref, o_ref):
        r, k = pl.program_id(0), pl.program_id(1)
        rows = r * R + lax.broadcasted_iota(jnp.int32, (R, KC), 0)
        P = (inv_ref[...] == rows).astype(jnp.bfloat16)
        contrib = jnp.dot(
            P, x_ref[...].astype(jnp.bfloat16),
            preferred_element_type=jnp.float32,
        ).astype(jnp.bfloat16)

        @pl.when(k == 0)
        def _():
            o_ref[...] = contrib

        @pl.when(k != 0)
        def _():
            o_ref[...] += contrib

    return pl.pallas_call(
        body,
        grid=(n_slots // R, T // KC),
        in_specs=[
            pl.BlockSpec((1, KC), lambda r, k: (0, k)),
            pl.BlockSpec((KC, D), lambda r, k: (k, 0)),
        ],
        out_specs=pl.BlockSpec((R, D), lambda r, k: (r, 0)),
        out_shape=jax.ShapeDtypeStruct((n_slots, D), jnp.bfloat16),
        compiler_params=_VMEM,
    )(inv_row, x)


def _gather_from_slots(y, inv_col, T):
    S, D = y.shape
    R, KC = 512, 1280

    def body(inv_ref, y_ref, o_ref):
        k = pl.program_id(1)
        cols = k * KC + lax.broadcasted_iota(jnp.int32, (R, KC), 1)
        P = (inv_ref[...] == cols).astype(jnp.bfloat16)
        contrib = jnp.dot(P, y_ref[...], preferred_element_type=jnp.float32)

        @pl.when(k == 0)
        def _():
            o_ref[...] = contrib

        @pl.when(k != 0)
        def _():
            o_ref[...] += contrib

    return pl.pallas_call(
        body,
        grid=(T // R, S // KC),
        in_specs=[
            pl.BlockSpec((R, 1), lambda r, k: (r, 0)),
            pl.BlockSpec((KC, D), lambda r, k: (k, 0)),
        ],
        out_specs=pl.BlockSpec((R, D), lambda r, k: (r, 0)),
        out_shape=jax.ShapeDtypeStruct((T, D), jnp.float32),
        compiler_params=_VMEM,
    )(inv_col, y)


def _a2a_exchange(buf, collective_id):

    def body(b_ref, o_ref, send_sem, recv_sem):
        my_x = lax.axis_index("x")
        my_y = lax.axis_index("y")
        my_z = lax.axis_index("z")
        partner = (1 - my_x, my_y, my_z)

        barrier = pltpu.get_barrier_semaphore()
        pl.semaphore_signal(
            barrier, inc=1, device_id=partner,
            device_id_type=pl.DeviceIdType.MESH,
        )
        pl.semaphore_wait(barrier, 1)

        rdma = pltpu.make_async_remote_copy(
            src_ref=b_ref,
            dst_ref=o_ref,
            send_sem=send_sem,
            recv_sem=recv_sem,
            device_id=partner,
            device_id_type=pl.DeviceIdType.MESH,
        )
        rdma.start()
        rdma.wait()

    return pl.pallas_call(
        body,
        out_shape=jax.ShapeDtypeStruct(buf.shape, buf.dtype),
        in_specs=[pl.BlockSpec(memory_space=pltpu.VMEM)],
        out_specs=pl.BlockSpec(memory_space=pltpu.VMEM),
        scratch_shapes=[pltpu.SemaphoreType.DMA, pltpu.SemaphoreType.DMA],
        compiler_params=pltpu.CompilerParams(collective_id=collective_id),
    )(buf)


def _expert_ffn(xcat, W1, W2):
    n_e, rows, d = xcat.shape
    f = W1.shape[2]
    nk = f // FC
    R = 640
    nr = rows // R

    def body(x_ref, w1_ref, w2_ref, o_ref):
        k = pl.program_id(2)
        h = jnp.maximum(
            jnp.dot(
                x_ref[0], w1_ref[0].astype(jnp.bfloat16),
                preferred_element_type=jnp.float32,
            ),
            0.0,
        ).astype(jnp.bfloat16)
        contrib = jnp.dot(
            h, w2_ref[0].astype(jnp.bfloat16),
            preferred_element_type=jnp.float32,
        )

        @pl.when(k == 0)
        def _():
            o_ref[0] = contrib

        @pl.when(k != 0)
        def _():
            o_ref[0] += contrib

    return pl.pallas_call(
        body,
        grid=(n_e, nr, nk),
        in_specs=[
            pl.BlockSpec((1, R, d), lambda e, r, k: (e, r, 0)),
            pl.BlockSpec((1, d, FC), lambda e, r, k: (e, 0, k)),
            pl.BlockSpec((1, FC, d), lambda e, r, k: (e, k, 0)),
        ],
        out_specs=pl.BlockSpec((1, R, d), lambda e, r, k: (e, r, 0)),
        out_shape=jax.ShapeDtypeStruct((n_e, rows, d), jnp.float32),
        compiler_params=_VMEM,
    )(xcat, W1, W2)


def kernel(x, assign, W1, W2):
    T, D = x.shape
    my_x = lax.axis_index("x")
    my_base = my_x * E_LOCAL

    oh = (assign[:, None] == jnp.arange(N_EXPERTS)[None, :]).astype(jnp.int32)
    rank = (oh * (jnp.cumsum(oh, axis=0) - oh)).sum(axis=1)
    slot_e = jnp.remainder(assign - my_base, N_EXPERTS)
    inv = (slot_e * C + rank).astype(jnp.int32)

    nloc = E_LOCAL * C
    gathered = _permute_to_slots(x, inv.reshape(1, T), 2 * nloc)
    local_buf = gathered[:nloc].reshape(E_LOCAL, C, D)
    send_buf = gathered[nloc:].reshape(E_LOCAL, C, D)

    recv_buf = _a2a_exchange(send_buf, collective_id=0)

    xcat = jnp.concatenate([local_buf, recv_buf], axis=1)
    y = _expert_ffn(xcat, W1, W2)
    y_local = y[:, :C].astype(jnp.bfloat16)
    y_remote = y[:, C:].astype(jnp.bfloat16)

    y_back = _a2a_exchange(y_remote, collective_id=1)

    y_all = jnp.concatenate([y_local, y_back]).reshape(-1, D)
    return _gather_from_slots(y_all, inv.reshape(T, 1), T)


# device time: 741586 ns/iter; 4.3603x vs baseline; 1.0589x over previous
import jax
import jax.numpy as jnp
from jax import lax
from jax.experimental import pallas as pl
from jax.experimental.pallas import tpu as pltpu

N_EXPERTS = 8
E_LOCAL = 4
C = 640
FC = 512
_VMEM = pltpu.CompilerParams(vmem_limit_bytes=56 * 1024 * 1024)


def _permute_to_slots(x, inv_row, n_slots):
    T, D = x.shape
    R, KC = 640, 1024

    def body(inv_ref, x_ref, o_ref):
        r, k = pl.program_id(0), pl.program_id(1)
        rows = r * R + lax.broadcasted_iota(jnp.int32, (R, KC), 0)
        P = (inv_ref[...] == rows).astype(jnp.bfloat16)
        contrib = jnp.dot(
            P, x_ref[...].astype(jnp.bfloat16),
            preferred_element_type=jnp.float32,
        ).astype(jnp.bfloat16)

        @pl.when(k == 0)
        def _():
            o_ref[...] = contrib

        @pl.when(k != 0)
        def _():
            o_ref[...] += contrib

    return pl.pallas_call(
        body,
        grid=(n_slots // R, T // KC),
        in_specs=[
            pl.BlockSpec((1, KC), lambda r, k: (0, k)),
            pl.BlockSpec((KC, D), lambda r, k: (k, 0)),
        ],
        out_specs=pl.BlockSpec((R, D), lambda r, k: (r, 0)),
        out_shape=jax.ShapeDtypeStruct((n_slots, D), jnp.bfloat16),
        compiler_params=_VMEM,
    )(inv_row, x)


def _gather_from_slots(y1, y2, inv_col, T):
    S = y1.shape[0] + y2.shape[0]
    D = y1.shape[1]
    R, KC = 512, 1280
    nk = S // KC

    def body(inv_ref, y1_ref, y2_ref, o_ref):
        k = pl.program_id(1)
        cols = k * KC + lax.broadcasted_iota(jnp.int32, (R, KC), 1)
        P = (inv_ref[...] == cols).astype(jnp.bfloat16)
        yk = jnp.where(k < nk // 2, y1_ref[...], y2_ref[...])
        contrib = jnp.dot(P, yk, preferred_element_type=jnp.float32)

        @pl.when(k == 0)
        def _():
            o_ref[...] = contrib

        @pl.when(k != 0)
        def _():
            o_ref[...] += contrib

    return pl.pallas_call(
        body,
        grid=(T // R, nk),
        in_specs=[
            pl.BlockSpec((R, 1), lambda r, k: (r, 0)),
            pl.BlockSpec((KC, D), lambda r, k: (jnp.minimum(k, 1), 0)),
            pl.BlockSpec((KC, D), lambda r, k: (jnp.maximum(k - 2, 0), 0)),
        ],
        out_specs=pl.BlockSpec((R, D), lambda r, k: (r, 0)),
        out_shape=jax.ShapeDtypeStruct((T, D), jnp.float32),
        compiler_params=_VMEM,
    )(inv_col, y1, y2)


def _a2a_exchange(buf, collective_id):

    def body(b_ref, o_ref, send_sem, recv_sem):
        my_x = lax.axis_index("x")
        my_y = lax.axis_index("y")
        my_z = lax.axis_index("z")
        partner = (1 - my_x, my_y, my_z)

        barrier = pltpu.get_barrier_semaphore()
        pl.semaphore_signal(
            barrier, inc=1, device_id=partner,
            device_id_type=pl.DeviceIdType.MESH,
        )
        pl.semaphore_wait(barrier, 1)

        rdma = pltpu.make_async_remote_copy(
            src_ref=b_ref,
            dst_ref=o_ref,
            send_sem=send_sem,
            recv_sem=recv_sem,
            device_id=partner,
            device_id_type=pl.DeviceIdType.MESH,
        )
        rdma.start()
        rdma.wait()

    return pl.pallas_call(
        body,
        out_shape=jax.ShapeDtypeStruct(buf.shape, buf.dtype),
        in_specs=[pl.BlockSpec(memory_space=pltpu.VMEM)],
        out_specs=pl.BlockSpec(memory_space=pltpu.VMEM),
        scratch_shapes=[pltpu.SemaphoreType.DMA, pltpu.SemaphoreType.DMA],
        compiler_params=pltpu.CompilerParams(collective_id=collective_id),
    )(buf)


def _expert_ffn(local_buf, recv_buf, W1, W2):
    n_e, c, d = local_buf.shape
    f = W1.shape[2]
    nk = f // FC

    def body(l_ref, r_ref, w1_ref, w2_ref, o1_ref, o2_ref, acc_ref):
        k = pl.program_id(1)
        xb = jnp.concatenate([l_ref[0], r_ref[0]], axis=0)
        h = jnp.maximum(
            jnp.dot(
                xb, w1_ref[0].astype(jnp.bfloat16),
                preferred_element_type=jnp.float32,
            ),
            0.0,
        ).astype(jnp.bfloat16)
        contrib = jnp.dot(
            h, w2_ref[0].astype(jnp.bfloat16),
            preferred_element_type=jnp.float32,
        )

        @pl.when(k == 0)
        def _():
            acc_ref[...] = contrib

        @pl.when(k != 0)
        def _():
            acc_ref[...] += contrib

        @pl.when(k == nk - 1)
        def _():
            o1_ref[0] = acc_ref[:c].astype(jnp.bfloat16)
            o2_ref[0] = acc_ref[c:].astype(jnp.bfloat16)

    return pl.pallas_call(
        body,
        grid=(n_e, nk),
        in_specs=[
            pl.BlockSpec((1, c, d), lambda e, k: (e, 0, 0)),
            pl.BlockSpec((1, c, d), lambda e, k: (e, 0, 0)),
            pl.BlockSpec((1, d, FC), lambda e, k: (e, 0, k)),
            pl.BlockSpec((1, FC, d), lambda e, k: (e, k, 0)),
        ],
        out_specs=[
            pl.BlockSpec((1, c, d), lambda e, k: (e, 0, 0)),
            pl.BlockSpec((1, c, d), lambda e, k: (e, 0, 0)),
        ],
        out_shape=[
            jax.ShapeDtypeStruct((n_e, c, d), jnp.bfloat16),
            jax.ShapeDtypeStruct((n_e, c, d), jnp.bfloat16),
        ],
        scratch_shapes=[pltpu.VMEM((2 * c, d), jnp.float32)],
        compiler_params=pltpu.CompilerParams(
            vmem_limit_bytes=60 * 1024 * 1024
        ),
    )(local_buf, recv_buf, W1, W2)


def kernel(x, assign, W1, W2):
    T, D = x.shape
    my_x = lax.axis_index("x")
    my_base = my_x * E_LOCAL

    oh = (assign[:, None] == jnp.arange(N_EXPERTS)[None, :]).astype(jnp.int32)
    rank = (oh * (jnp.cumsum(oh, axis=0) - oh)).sum(axis=1)
    slot_e = jnp.remainder(assign - my_base, N_EXPERTS)
    inv = (slot_e * C + rank).astype(jnp.int32)

    nloc = E_LOCAL * C
    gathered = _permute_to_slots(x, inv.reshape(1, T), 2 * nloc)
    local_buf = gathered[:nloc].reshape(E_LOCAL, C, D)
    send_buf = gathered[nloc:].reshape(E_LOCAL, C, D)

    recv_buf = _a2a_exchange(send_buf, collective_id=0)

    y_local, y_remote = _expert_ffn(local_buf, recv_buf, W1, W2)

    y_back = _a2a_exchange(y_remote, collective_id=1)

    return _gather_from_slots(
        y_local.reshape(-1, D), y_back.reshape(-1, D), inv.reshape(T, 1), T
    )
